# baseline (speedup 1.0000x reference)
import functools

import jax
import jax.numpy as jnp
from jax.experimental import pallas as pl
from jax.experimental.pallas import tpu as pltpu


def _body(nk, l_ref, fk_ref, fm_ref, w1t_ref, w2t_ref, b_ref, out_ref, acc_ref):
    k = pl.program_id(1)

    @pl.when(k == 0)
    def _():
        acc_ref[...] = jnp.zeros_like(acc_ref)

    acc_ref[...] += jnp.dot(
        l_ref[...], fk_ref[...], preferred_element_type=jnp.float32
    )

    @pl.when(k == nk - 1)
    def _():
        x = acc_ref[...]
        f = fm_ref[...]
        out_ref[...] = (
            jnp.dot(f + x, w1t_ref[...], preferred_element_type=jnp.float32)
            + jnp.dot(x * f, w2t_ref[...], preferred_element_type=jnp.float32)
            + b_ref[...]
        )


def kernel(lap_matrix, eye_matrix, features, W1, b1, W2, b2):
    n, d = features.shape
    bm = min(512, n)
    bk = min(2048, n)
    nm = n // bm
    nk = n // bk

    bias = (b1 + b2).reshape(1, d)

    in_specs = [
            pl.BlockSpec((bm, bk), lambda i, k: (i, k)),
            pl.BlockSpec((bk, d), lambda i, k: (k, 0)),
            pl.BlockSpec((bm, d), lambda i, k: (i, 0)),
            pl.BlockSpec((d, d), lambda i, k: (0, 0)),
            pl.BlockSpec((d, d), lambda i, k: (0, 0)),
            pl.BlockSpec((1, d), lambda i, k: (0, 0)),
    ]

    return pl.pallas_call(
        functools.partial(_body, nk),
        grid=(nm, nk),
        in_specs=in_specs,
        out_specs=pl.BlockSpec((bm, d), lambda i, k: (i, 0)),
        out_shape=jax.ShapeDtypeStruct((n, d), jnp.float32),
        scratch_shapes=[pltpu.VMEM((bm, d), jnp.float32)],
        compiler_params=pltpu.CompilerParams(
            dimension_semantics=("parallel", "arbitrary"),
        ),
    )(lap_matrix, features, features, W1.T, W2.T, bias)


# speedup vs baseline: 1.0007x; 1.0007x over previous
import functools

import jax
import jax.numpy as jnp
from jax.experimental import pallas as pl
from jax.experimental.pallas import tpu as pltpu


def _body(nk, l_ref, fk_ref, fm_ref, w1t_ref, w2t_ref, b_ref, out_ref, acc_ref):
    k = pl.program_id(1)

    @pl.when(k == 0)
    def _():
        acc_ref[...] = jnp.zeros_like(acc_ref)

    acc_ref[...] += jnp.dot(
        l_ref[...].astype(jnp.bfloat16),
        fk_ref[...].astype(jnp.bfloat16),
        preferred_element_type=jnp.float32,
    )

    @pl.when(k == nk - 1)
    def _():
        x = acc_ref[...]
        f = fm_ref[...]
        out_ref[...] = (
            jnp.dot(f + x, w1t_ref[...], preferred_element_type=jnp.float32)
            + jnp.dot(x * f, w2t_ref[...], preferred_element_type=jnp.float32)
            + b_ref[...]
        )


def kernel(lap_matrix, eye_matrix, features, W1, b1, W2, b2):
    n, d = features.shape
    bm = min(512, n)
    bk = min(2048, n)
    nm = n // bm
    nk = n // bk

    bias = (b1 + b2).reshape(1, d)

    in_specs = [
            pl.BlockSpec((bm, bk), lambda i, k: (i, k)),
            pl.BlockSpec((bk, d), lambda i, k: (k, 0)),
            pl.BlockSpec((bm, d), lambda i, k: (i, 0)),
            pl.BlockSpec((d, d), lambda i, k: (0, 0)),
            pl.BlockSpec((d, d), lambda i, k: (0, 0)),
            pl.BlockSpec((1, d), lambda i, k: (0, 0)),
    ]

    return pl.pallas_call(
        functools.partial(_body, nk),
        grid=(nm, nk),
        in_specs=in_specs,
        out_specs=pl.BlockSpec((bm, d), lambda i, k: (i, 0)),
        out_shape=jax.ShapeDtypeStruct((n, d), jnp.float32),
        scratch_shapes=[pltpu.VMEM((bm, d), jnp.float32)],
        compiler_params=pltpu.CompilerParams(
            dimension_semantics=("parallel", "arbitrary"),
        ),
    )(lap_matrix, features, features, W1.T, W2.T, bias)


# speedup vs baseline: 1.3255x; 1.3246x over previous
import functools

import jax
import jax.numpy as jnp
from jax.experimental import pallas as pl
from jax.experimental.pallas import tpu as pltpu


def _body(nk, l_ref, fk_ref, fm_ref, w1t_ref, w2t_ref, b_ref, out_ref, acc_ref):
    k = pl.program_id(1)

    @pl.when(k == 0)
    def _():
        acc_ref[...] = jnp.zeros_like(acc_ref)

    acc_ref[...] += jnp.dot(
        l_ref[...].astype(jnp.bfloat16),
        fk_ref[...].astype(jnp.bfloat16),
        preferred_element_type=jnp.float32,
    )

    @pl.when(k == nk - 1)
    def _():
        x = acc_ref[...]
        f = fm_ref[...]
        out_ref[...] = (
            jnp.dot(f + x, w1t_ref[...], preferred_element_type=jnp.float32)
            + jnp.dot(x * f, w2t_ref[...], preferred_element_type=jnp.float32)
            + b_ref[...]
        )


def kernel(lap_matrix, eye_matrix, features, W1, b1, W2, b2):
    n, d = features.shape
    bm = min(256, n)
    bk = n
    nm = n // bm
    nk = n // bk

    bias = (b1 + b2).reshape(1, d)

    in_specs = [
            pl.BlockSpec((bm, bk), lambda i, k: (i, k)),
            pl.BlockSpec((bk, d), lambda i, k: (k, 0)),
            pl.BlockSpec((bm, d), lambda i, k: (i, 0)),
            pl.BlockSpec((d, d), lambda i, k: (0, 0)),
            pl.BlockSpec((d, d), lambda i, k: (0, 0)),
            pl.BlockSpec((1, d), lambda i, k: (0, 0)),
    ]

    return pl.pallas_call(
        functools.partial(_body, nk),
        grid=(nm, nk),
        in_specs=in_specs,
        out_specs=pl.BlockSpec((bm, d), lambda i, k: (i, 0)),
        out_shape=jax.ShapeDtypeStruct((n, d), jnp.float32),
        scratch_shapes=[pltpu.VMEM((bm, d), jnp.float32)],
        compiler_params=pltpu.CompilerParams(
            dimension_semantics=("parallel", "arbitrary"),
        ),
    )(lap_matrix, features, features, W1.T, W2.T, bias)


# speedup vs baseline: 1.3296x; 1.0031x over previous
import jax
import jax.numpy as jnp
from jax.experimental import pallas as pl
from jax.experimental.pallas import tpu as pltpu


def _body(l_ref, f_ref, fm_ref, w1t_ref, w2t_ref, b_ref, out_ref):
    x = jnp.dot(
        l_ref[...].astype(jnp.bfloat16),
        f_ref[...].astype(jnp.bfloat16),
        preferred_element_type=jnp.float32,
    )
    f = fm_ref[...]
    out_ref[...] = (
        jnp.dot(f + x, w1t_ref[...], preferred_element_type=jnp.float32)
        + jnp.dot(x * f, w2t_ref[...], preferred_element_type=jnp.float32)
        + b_ref[...]
    )


def kernel(lap_matrix, eye_matrix, features, W1, b1, W2, b2):
    n, d = features.shape
    bm = min(256, n)
    nm = n // bm

    bias = (b1 + b2).reshape(1, d)

    in_specs = [
        pl.BlockSpec((bm, n), lambda i: (i, 0)),
        pl.BlockSpec((n, d), lambda i: (0, 0)),
        pl.BlockSpec((bm, d), lambda i: (i, 0)),
        pl.BlockSpec((d, d), lambda i: (0, 0)),
        pl.BlockSpec((d, d), lambda i: (0, 0)),
        pl.BlockSpec((1, d), lambda i: (0, 0)),
    ]

    return pl.pallas_call(
        _body,
        grid=(nm,),
        in_specs=in_specs,
        out_specs=pl.BlockSpec((bm, d), lambda i: (i, 0)),
        out_shape=jax.ShapeDtypeStruct((n, d), jnp.float32),
        compiler_params=pltpu.CompilerParams(
            dimension_semantics=("arbitrary",),
        ),
    )(lap_matrix, features, features, W1.T, W2.T, bias)
